# baseline (device time: 11111 ns/iter reference)
import jax
import jax.numpy as jnp
from jax import lax
from jax.experimental import pallas as pl
from jax.experimental.pallas import tpu as pltpu

N_DEV = 4


def kernel(x):
    m, n = x.shape
    h = m // 2
    q = m // 4

    def body(x_ref, out_ref, send_sems, recv_sems):
        my = lax.axis_index("i")
        left = (my - 1) % N_DEV
        right = (my + 1) % N_DEV
        opp = (my + 2) % N_DEV

        barrier_sem = pltpu.get_barrier_semaphore()
        for nbr in [left, right]:
            pl.semaphore_signal(
                barrier_sem, inc=1,
                device_id=(nbr,), device_id_type=pl.DeviceIdType.MESH,
            )
        pl.semaphore_wait(barrier_sem, 2)

        def copy(src, dst, sem, dev):
            return pltpu.make_async_remote_copy(
                src_ref=src, dst_ref=dst,
                send_sem=send_sems.at[sem], recv_sem=recv_sems.at[sem],
                device_id=(dev,), device_id_type=pl.DeviceIdType.MESH,
            )

        def row(base, off, size):
            return out_ref.at[pl.ds(base * m + off, size)]

        sends = [
            copy(x_ref.at[pl.ds(0, q)], row(my, 0, q), 0, right),
            copy(x_ref.at[pl.ds(q, q)], row(my, q, q), 1, right),
            copy(x_ref.at[pl.ds(h, q)], row(my, h, q), 2, left),
            copy(x_ref.at[pl.ds(h + q, q)], row(my, h + q, q), 3, left),
            copy(x_ref.at[pl.ds(h, h)], row(my, h, h), 4, right),
            copy(x_ref.at[pl.ds(0, h)], row(my, 0, h), 5, left),
        ]
        for s in sends:
            s.start()

        out_ref[pl.ds(my * m, m), :] = x_ref[:, :]

        fwds = []
        for i, (src_base, off, sem_in, sem_out, dev) in enumerate([
            (left, 0, 0, 6, right),
            (left, q, 1, 7, right),
            (right, h, 2, 8, left),
            (right, h + q, 3, 9, left),
        ]):
            copy(row(src_base, off, q), row(src_base, off, q),
                 sem_in, dev).wait_recv()
            f = copy(row(src_base, off, q), row(src_base, off, q),
                     sem_out, dev)
            f.start()
            fwds.append(f)

        copy(row(left, h, h), row(left, h, h), 4, right).wait_recv()
        copy(row(right, 0, h), row(right, 0, h), 5, left).wait_recv()
        copy(row(opp, 0, q), row(opp, 0, q), 6, right).wait_recv()
        copy(row(opp, q, q), row(opp, q, q), 7, right).wait_recv()
        copy(row(opp, h, q), row(opp, h, q), 8, left).wait_recv()
        copy(row(opp, h + q, q), row(opp, h + q, q), 9, left).wait_recv()

        for s in sends + fwds:
            s.wait_send()

    return pl.pallas_call(
        body,
        out_shape=jax.ShapeDtypeStruct((N_DEV * m, n), x.dtype),
        in_specs=[pl.BlockSpec(memory_space=pltpu.VMEM)],
        out_specs=pl.BlockSpec(memory_space=pltpu.VMEM),
        scratch_shapes=[
            pltpu.SemaphoreType.DMA((10,)),
            pltpu.SemaphoreType.DMA((10,)),
        ],
        compiler_params=pltpu.CompilerParams(collective_id=0),
    )(x)


# device time: 11079 ns/iter; 1.0029x vs baseline; 1.0029x over previous
import jax
import jax.numpy as jnp
from jax import lax
from jax.experimental import pallas as pl
from jax.experimental.pallas import tpu as pltpu

N_DEV = 4


def kernel(x):
    m, n = x.shape
    h = m // 2

    def body(x_ref, out_ref, send_sems, recv_sems):
        my = lax.axis_index("i")
        left = (my - 1) % N_DEV
        right = (my + 1) % N_DEV
        opp = (my + 2) % N_DEV

        barrier_sem = pltpu.get_barrier_semaphore()
        for nbr in [left, right]:
            pl.semaphore_signal(
                barrier_sem, inc=1,
                device_id=(nbr,), device_id_type=pl.DeviceIdType.MESH,
            )
        pl.semaphore_wait(barrier_sem, 2)

        def copy(src, dst, sem, dev):
            return pltpu.make_async_remote_copy(
                src_ref=src, dst_ref=dst,
                send_sem=send_sems.at[sem], recv_sem=recv_sems.at[sem],
                device_id=(dev,), device_id_type=pl.DeviceIdType.MESH,
            )

        my_top = out_ref.at[pl.ds(my * m, h)]
        my_bot = out_ref.at[pl.ds(my * m + h, h)]

        s_top_r = copy(x_ref.at[pl.ds(0, h)], my_top, 0, right)
        s_bot_l = copy(x_ref.at[pl.ds(h, h)], my_bot, 1, left)
        s_bot_r = copy(x_ref.at[pl.ds(h, h)], my_bot, 2, right)
        s_top_l = copy(x_ref.at[pl.ds(0, h)], my_top, 3, left)
        s_top_r.start()
        s_bot_l.start()
        s_bot_r.start()
        s_top_l.start()

        out_ref[pl.ds(my * m, m), :] = x_ref[:, :]

        left_top = out_ref.at[pl.ds(left * m, h)]
        left_bot = out_ref.at[pl.ds(left * m + h, h)]
        right_top = out_ref.at[pl.ds(right * m, h)]
        right_bot = out_ref.at[pl.ds(right * m + h, h)]
        opp_top = out_ref.at[pl.ds(opp * m, h)]
        opp_bot = out_ref.at[pl.ds(opp * m + h, h)]

        copy(left_top, left_top, 0, right).wait_recv()
        f_r = copy(left_top, left_top, 4, right)
        f_r.start()

        copy(right_bot, right_bot, 1, left).wait_recv()
        f_l = copy(right_bot, right_bot, 5, left)
        f_l.start()

        copy(left_bot, left_bot, 2, right).wait_recv()
        copy(right_top, right_top, 3, left).wait_recv()
        copy(opp_top, opp_top, 4, right).wait_recv()
        copy(opp_bot, opp_bot, 5, left).wait_recv()

        s_top_r.wait_send()
        s_bot_l.wait_send()
        s_bot_r.wait_send()
        s_top_l.wait_send()
        f_r.wait_send()
        f_l.wait_send()

    return pl.pallas_call(
        body,
        out_shape=jax.ShapeDtypeStruct((N_DEV * m, n), x.dtype),
        in_specs=[pl.BlockSpec(memory_space=pltpu.VMEM)],
        out_specs=pl.BlockSpec(memory_space=pltpu.VMEM),
        scratch_shapes=[
            pltpu.SemaphoreType.DMA((6,)),
            pltpu.SemaphoreType.DMA((6,)),
        ],
        compiler_params=pltpu.CompilerParams(collective_id=0),
    )(x)
